# baseline (speedup 1.0000x reference)
import functools

import jax
import jax.numpy as jnp
from jax.experimental import pallas as pl
from jax.experimental.pallas import tpu as pltpu

_MASK_RATIO = 0.25


def _body(x_ref, vis_ref, msk_ref, mask_ref, sem_v, sem_m, *, nv, num_mask):
    b, s = mask_ref.shape
    copies = []
    for bi in range(b):
        cv = pltpu.make_async_copy(x_ref.at[bi, pl.ds(0, nv)], vis_ref.at[bi], sem_v)
        cv.start()
        cm = pltpu.make_async_copy(x_ref.at[bi, pl.ds(nv, num_mask)], msk_ref.at[bi], sem_m)
        cm.start()
        copies += [cv, cm]
    col = jax.lax.broadcasted_iota(jnp.int32, (b, s), 1)
    mask_ref[...] = col >= nv
    for c in copies:
        c.wait()


def kernel(x):
    b, s, f = x.shape
    num_mask = int(s * _MASK_RATIO)
    nv = s - num_mask

    visible, masked, mask = pl.pallas_call(
        functools.partial(_body, nv=nv, num_mask=num_mask),
        in_specs=[pl.BlockSpec(memory_space=pl.ANY)],
        out_specs=[
            pl.BlockSpec(memory_space=pl.ANY),
            pl.BlockSpec(memory_space=pl.ANY),
            pl.BlockSpec((b, s), lambda: (0, 0)),
        ],
        out_shape=[
            jax.ShapeDtypeStruct((b, nv, f), x.dtype),
            jax.ShapeDtypeStruct((b, num_mask, f), x.dtype),
            jax.ShapeDtypeStruct((b, s), jnp.bool_),
        ],
        scratch_shapes=[
            pltpu.SemaphoreType.DMA,
            pltpu.SemaphoreType.DMA,
        ],
    )(x)

    return visible, masked, mask


# speedup vs baseline: 34.7735x; 34.7735x over previous
import functools

import jax
import jax.numpy as jnp
from jax import lax
from jax.experimental import pallas as pl
from jax.experimental.pallas import tpu as pltpu
from jax.experimental.pallas import tpu_sc as plsc

_MASK_RATIO = 0.25
_NC = 2
_NS = 16


def _mask_body(mask_ref):
    b, s = mask_ref.shape
    nv = s - int(s * _MASK_RATIO)
    col = jax.lax.broadcasted_iota(jnp.int32, (b, s), 1)
    mask_ref[...] = col >= nv


def kernel(x):
    b, s, f = x.shape
    num_mask = int(s * _MASK_RATIO)
    nv = s - num_mask

    nw = _NC * _NS
    rpw = (b * s) // nw
    wpb = s // rpw
    vis_w = nv // rpw
    ch = 16
    nch = rpw // ch

    mesh = plsc.VectorSubcoreMesh(core_axis_name="c", subcore_axis_name="s")

    @functools.partial(
        pl.kernel,
        mesh=mesh,
        out_type=[
            jax.ShapeDtypeStruct((b, nv, f), x.dtype),
            jax.ShapeDtypeStruct((b, num_mask, f), x.dtype),
        ],
        scratch_types=[
            pltpu.VMEM((ch, f), jnp.float32),
            pltpu.VMEM((ch, f), jnp.float32),
            pltpu.SemaphoreType.DMA,
            pltpu.SemaphoreType.DMA,
            pltpu.SemaphoreType.DMA,
            pltpu.SemaphoreType.DMA,
        ],
    )
    def sc_copy(x_hbm, vis_hbm, msk_hbm, buf0, buf1, si0, si1, so0, so1):
        wid = lax.axis_index("s") * _NC + lax.axis_index("c")
        bi = wid // wpb
        k = wid % wpb
        r0 = k * rpw

        def side(dst, dbase):
            bufs = (buf0, buf1)
            sin = (si0, si1)
            sout = (so0, so1)
            cin = [None, None]
            cout = [None, None]
            cin[0] = pltpu.async_copy(x_hbm.at[bi, pl.ds(r0, ch)], buf0, si0)
            for c in range(nch):
                p = c % 2
                cin[p].wait()
                if c + 1 < nch:
                    q = (c + 1) % 2
                    if cout[q] is not None:
                        cout[q].wait()
                    cin[q] = pltpu.async_copy(
                        x_hbm.at[bi, pl.ds(r0 + (c + 1) * ch, ch)], bufs[q], sin[q]
                    )
                cout[p] = pltpu.async_copy(
                    bufs[p], dst.at[bi, pl.ds(dbase + c * ch, ch)], sout[p]
                )
            cout[(nch - 1) % 2].wait()
            if nch > 1:
                cout[nch % 2].wait()

        @pl.when(k < vis_w)
        def _():
            side(vis_hbm, r0)

        @pl.when(k >= vis_w)
        def _():
            side(msk_hbm, r0 - nv)

    visible, masked = sc_copy(x)

    mask = pl.pallas_call(
        _mask_body,
        out_shape=jax.ShapeDtypeStruct((b, s), jnp.bool_),
    )()

    return visible, masked, mask


# speedup vs baseline: 39.6983x; 1.1416x over previous
import functools

import jax
import jax.numpy as jnp
from jax import lax
from jax.experimental import pallas as pl
from jax.experimental.pallas import tpu as pltpu
from jax.experimental.pallas import tpu_sc as plsc

_MASK_RATIO = 0.25
_NC = 2
_NS = 16


def _vis_body(x_ref, vis_ref, mask_ref, *, nv):
    i = pl.program_id(0)
    j = pl.program_id(1)

    @pl.when(jnp.logical_and(i == 0, j == 0))
    def _():
        b, s = mask_ref.shape
        col = jax.lax.broadcasted_iota(jnp.int32, (b, s), 1)
        mask_ref[...] = col >= nv

    vis_ref[...] = x_ref[...]


def _make_sc_copy(b, s, f, row0, rows, dtype):
    nw = _NC * _NS
    rpw = (b * rows) // nw
    wpb = rows // rpw
    ch = min(16, rpw)
    nch = rpw // ch

    mesh = plsc.VectorSubcoreMesh(core_axis_name="c", subcore_axis_name="s")

    @functools.partial(
        pl.kernel,
        mesh=mesh,
        out_type=jax.ShapeDtypeStruct((b, rows, f), dtype),
        scratch_types=[
            pltpu.VMEM((ch, f), jnp.float32),
            pltpu.VMEM((ch, f), jnp.float32),
            pltpu.SemaphoreType.DMA,
            pltpu.SemaphoreType.DMA,
            pltpu.SemaphoreType.DMA,
            pltpu.SemaphoreType.DMA,
        ],
    )
    def sc_copy(x_hbm, out_hbm, buf0, buf1, si0, si1, so0, so1):
        wid = lax.axis_index("s") * _NC + lax.axis_index("c")
        bi = wid // wpb
        k = wid % wpb
        src0 = row0 + k * rpw
        dst0 = k * rpw

        bufs = (buf0, buf1)
        sin = (si0, si1)
        sout = (so0, so1)
        cin = [None, None]
        cout = [None, None]
        cin[0] = pltpu.async_copy(x_hbm.at[bi, pl.ds(src0, ch)], buf0, si0)
        for c in range(nch):
            p = c % 2
            cin[p].wait()
            if c + 1 < nch:
                q = (c + 1) % 2
                if cout[q] is not None:
                    cout[q].wait()
                cin[q] = pltpu.async_copy(
                    x_hbm.at[bi, pl.ds(src0 + (c + 1) * ch, ch)], bufs[q], sin[q]
                )
            cout[p] = pltpu.async_copy(
                bufs[p], out_hbm.at[bi, pl.ds(dst0 + c * ch, ch)], sout[p]
            )
        cout[(nch - 1) % 2].wait()
        if nch > 1:
            cout[nch % 2].wait()

    return sc_copy


def kernel(x):
    b, s, f = x.shape
    num_mask = int(s * _MASK_RATIO)
    nv = s - num_mask

    masked = _make_sc_copy(b, s, f, nv, num_mask, x.dtype)(x)

    bs = 1024
    visible, mask = pl.pallas_call(
        functools.partial(_vis_body, nv=nv),
        grid=(b, nv // bs),
        in_specs=[pl.BlockSpec((1, bs, f), lambda i, j: (i, j, 0))],
        out_specs=[
            pl.BlockSpec((1, bs, f), lambda i, j: (i, j, 0)),
            pl.BlockSpec((b, s), lambda i, j: (0, 0)),
        ],
        out_shape=[
            jax.ShapeDtypeStruct((b, nv, f), x.dtype),
            jax.ShapeDtypeStruct((b, s), jnp.bool_),
        ],
    )(x)

    return visible, masked, mask


# speedup vs baseline: 39.7031x; 1.0001x over previous
import functools

import jax
import jax.numpy as jnp
from jax import lax
from jax.experimental import pallas as pl
from jax.experimental.pallas import tpu as pltpu
from jax.experimental.pallas import tpu_sc as plsc

_MASK_RATIO = 0.25
_NC = 2
_NS = 16


def _vis_body(x_ref, vis_ref, mask_ref, *, nv):
    i = pl.program_id(0)
    j = pl.program_id(1)

    @pl.when(jnp.logical_and(i == 0, j == 0))
    def _():
        b, s = mask_ref.shape
        col = jax.lax.broadcasted_iota(jnp.int32, (b, s), 1)
        mask_ref[...] = col >= nv

    vis_ref[...] = x_ref[...]


def _make_sc_copy(b, s, f, row0, rows, dtype):
    nw = _NC * _NS
    rpw = (b * rows) // nw
    wpb = rows // rpw
    ch = min(16, rpw)
    nch = rpw // ch

    mesh = plsc.VectorSubcoreMesh(core_axis_name="c", subcore_axis_name="s")

    @functools.partial(
        pl.kernel,
        mesh=mesh,
        out_type=jax.ShapeDtypeStruct((b, rows, f), dtype),
        scratch_types=[
            pltpu.VMEM((ch, f), jnp.float32),
            pltpu.VMEM((ch, f), jnp.float32),
            pltpu.SemaphoreType.DMA,
            pltpu.SemaphoreType.DMA,
            pltpu.SemaphoreType.DMA,
            pltpu.SemaphoreType.DMA,
        ],
    )
    def sc_copy(x_hbm, out_hbm, buf0, buf1, si0, si1, so0, so1):
        wid = lax.axis_index("s") * _NC + lax.axis_index("c")
        bi = wid // wpb
        k = wid % wpb
        src0 = row0 + k * rpw
        dst0 = k * rpw

        bufs = (buf0, buf1)
        sin = (si0, si1)
        sout = (so0, so1)
        cin = [None, None]
        cout = [None, None]
        cin[0] = pltpu.async_copy(x_hbm.at[bi, pl.ds(src0, ch)], buf0, si0)
        for c in range(nch):
            p = c % 2
            cin[p].wait()
            if c + 1 < nch:
                q = (c + 1) % 2
                if cout[q] is not None:
                    cout[q].wait()
                cin[q] = pltpu.async_copy(
                    x_hbm.at[bi, pl.ds(src0 + (c + 1) * ch, ch)], bufs[q], sin[q]
                )
            cout[p] = pltpu.async_copy(
                bufs[p], out_hbm.at[bi, pl.ds(dst0 + c * ch, ch)], sout[p]
            )
        cout[(nch - 1) % 2].wait()
        if nch > 1:
            cout[nch % 2].wait()

    return sc_copy


def kernel(x):
    b, s, f = x.shape
    num_mask = int(s * _MASK_RATIO)
    nv = s - num_mask

    bs = 1024
    visible, mask = pl.pallas_call(
        functools.partial(_vis_body, nv=nv),
        grid=(b, nv // bs),
        in_specs=[pl.BlockSpec((1, bs, f), lambda i, j: (i, j, 0))],
        out_specs=[
            pl.BlockSpec((1, bs, f), lambda i, j: (i, j, 0)),
            pl.BlockSpec((b, s), lambda i, j: (0, 0)),
        ],
        out_shape=[
            jax.ShapeDtypeStruct((b, nv, f), x.dtype),
            jax.ShapeDtypeStruct((b, s), jnp.bool_),
        ],
    )(x)

    masked = _make_sc_copy(b, s, f, nv, num_mask, x.dtype)(x)

    return visible, masked, mask


# speedup vs baseline: 39.7247x; 1.0005x over previous
import functools

import jax
import jax.numpy as jnp
from jax import lax
from jax.experimental import pallas as pl
from jax.experimental.pallas import tpu as pltpu
from jax.experimental.pallas import tpu_sc as plsc

_MASK_RATIO = 0.25
_NC = 2
_NS = 16


def _vis_body(x_ref, vis_ref, mask_ref, *, nv):
    i = pl.program_id(0)
    j = pl.program_id(1)

    @pl.when(jnp.logical_and(i == 0, j == 0))
    def _():
        b, s = mask_ref.shape
        col = jax.lax.broadcasted_iota(jnp.int32, (b, s), 1)
        mask_ref[...] = col >= nv

    vis_ref[...] = x_ref[...]


def _make_sc_copy(b, s, f, row0, rows, dtype):
    nw = _NC * _NS
    rpw = (b * rows) // nw
    wpb = rows // rpw
    ch = min(16, rpw)
    nch = rpw // ch

    mesh = plsc.VectorSubcoreMesh(core_axis_name="c", subcore_axis_name="s")

    @functools.partial(
        pl.kernel,
        mesh=mesh,
        out_type=jax.ShapeDtypeStruct((b, rows, f), dtype),
        scratch_types=[
            pltpu.VMEM((ch, f), jnp.float32),
            pltpu.VMEM((ch, f), jnp.float32),
            pltpu.SemaphoreType.DMA,
            pltpu.SemaphoreType.DMA,
            pltpu.SemaphoreType.DMA,
            pltpu.SemaphoreType.DMA,
        ],
        cost_estimate=pl.CostEstimate(
            flops=0, bytes_accessed=2 * b * rows * f * 4, transcendentals=0
        ),
    )
    def sc_copy(x_hbm, out_hbm, buf0, buf1, si0, si1, so0, so1):
        wid = lax.axis_index("s") * _NC + lax.axis_index("c")
        bi = wid // wpb
        k = wid % wpb
        src0 = row0 + k * rpw
        dst0 = k * rpw

        bufs = (buf0, buf1)
        sin = (si0, si1)
        sout = (so0, so1)
        cin = [None, None]
        cout = [None, None]
        cin[0] = pltpu.async_copy(x_hbm.at[bi, pl.ds(src0, ch)], buf0, si0)
        for c in range(nch):
            p = c % 2
            cin[p].wait()
            if c + 1 < nch:
                q = (c + 1) % 2
                if cout[q] is not None:
                    cout[q].wait()
                cin[q] = pltpu.async_copy(
                    x_hbm.at[bi, pl.ds(src0 + (c + 1) * ch, ch)], bufs[q], sin[q]
                )
            cout[p] = pltpu.async_copy(
                bufs[p], out_hbm.at[bi, pl.ds(dst0 + c * ch, ch)], sout[p]
            )
        cout[(nch - 1) % 2].wait()
        if nch > 1:
            cout[nch % 2].wait()

    return sc_copy


def kernel(x):
    b, s, f = x.shape
    num_mask = int(s * _MASK_RATIO)
    nv = s - num_mask

    bs = 1024
    visible, mask = pl.pallas_call(
        functools.partial(_vis_body, nv=nv),
        grid=(b, nv // bs),
        in_specs=[pl.BlockSpec((1, bs, f), lambda i, j: (i, j, 0))],
        out_specs=[
            pl.BlockSpec((1, bs, f), lambda i, j: (i, j, 0)),
            pl.BlockSpec((b, s), lambda i, j: (0, 0)),
        ],
        out_shape=[
            jax.ShapeDtypeStruct((b, nv, f), x.dtype),
            jax.ShapeDtypeStruct((b, s), jnp.bool_),
        ],
        cost_estimate=pl.CostEstimate(
            flops=0, bytes_accessed=2 * b * nv * f * 4, transcendentals=0
        ),
    )(x)

    masked = _make_sc_copy(b, s, f, nv, num_mask, x.dtype)(x)

    return visible, masked, mask
